# baseline (speedup 1.0000x reference)
import jax
import jax.numpy as jnp
from jax.experimental import pallas as pl
from jax.experimental.pallas import tpu as pltpu

N = 10000
TM = 400


def _pass_a(data_ref, w1_ref, s1_ref):
    s1_ref[...] = jnp.dot(data_ref[...], w1_ref[...],
                          preferred_element_type=jnp.float32)


def _pass_b(adj_ref, s1_ref, b1_ref, w2_ref, s2_ref):
    p = jnp.dot(adj_ref[...], s1_ref[...],
                preferred_element_type=jnp.float32)
    h = jnp.maximum(p + b1_ref[...], 0.0)
    s2_ref[...] = jnp.dot(h, w2_ref[...],
                          preferred_element_type=jnp.float32)


def _pass_c(adj_ref, s2_ref, b2_ref, out_ref):
    z = jnp.dot(adj_ref[...], s2_ref[...],
                preferred_element_type=jnp.float32) + b2_ref[...]
    m = jnp.max(z, axis=1, keepdims=True)
    lse = jnp.log(jnp.sum(jnp.exp(z - m), axis=1, keepdims=True)) + m
    out_ref[...] = z - lse


def kernel(data, adj, W1, b1, W2, b2):
    in_feat = data.shape[1]
    hid = W1.shape[1]
    nout = W2.shape[1]
    kpad = ((in_feat + 127) // 128) * 128 - in_feat
    data_p = jnp.pad(data, ((0, 0), (0, kpad)))
    w1_p = jnp.pad(W1, ((0, kpad), (0, 0)))
    kp = in_feat + kpad
    b1r = b1.reshape(1, hid)
    b2r = b2.reshape(1, nout)

    s1 = pl.pallas_call(
        _pass_a,
        grid=(5,),
        in_specs=[
            pl.BlockSpec((N // 5, kp), lambda i: (i, 0)),
            pl.BlockSpec((kp, hid), lambda i: (0, 0)),
        ],
        out_specs=pl.BlockSpec((N // 5, hid), lambda i: (i, 0)),
        out_shape=jax.ShapeDtypeStruct((N, hid), jnp.float32),
        compiler_params=pltpu.CompilerParams(
            dimension_semantics=("arbitrary",)),
    )(data_p, w1_p)

    grid = (N // TM,)
    s2 = pl.pallas_call(
        _pass_b,
        grid=grid,
        in_specs=[
            pl.BlockSpec((TM, N), lambda i: (i, 0)),
            pl.BlockSpec((N, hid), lambda i: (0, 0)),
            pl.BlockSpec((1, hid), lambda i: (0, 0)),
            pl.BlockSpec((hid, nout), lambda i: (0, 0)),
        ],
        out_specs=pl.BlockSpec((TM, nout), lambda i: (i, 0)),
        out_shape=jax.ShapeDtypeStruct((N, nout), jnp.float32),
        compiler_params=pltpu.CompilerParams(
            dimension_semantics=("arbitrary",)),
    )(adj, s1, b1r, W2)

    out = pl.pallas_call(
        _pass_c,
        grid=grid,
        in_specs=[
            pl.BlockSpec((TM, N), lambda i: (i, 0)),
            pl.BlockSpec((N, nout), lambda i: (0, 0)),
            pl.BlockSpec((1, nout), lambda i: (0, 0)),
        ],
        out_specs=pl.BlockSpec((TM, nout), lambda i: (i, 0)),
        out_shape=jax.ShapeDtypeStruct((N, nout), jnp.float32),
        compiler_params=pltpu.CompilerParams(
            dimension_semantics=("arbitrary",)),
    )(adj, s2, b2r)

    return out


# speedup vs baseline: 1.8096x; 1.8096x over previous
import jax
import jax.numpy as jnp
from jax.experimental import pallas as pl
from jax.experimental.pallas import tpu as pltpu

N = 10000
TM = 400


def _pass_a(data_ref, w1_ref, s1_ref):
    s1_ref[...] = jnp.dot(data_ref[...], w1_ref[...],
                          preferred_element_type=jnp.float32)


def _pass_b(adj_ref, s1_ref, b1_ref, w2_ref, s2_ref):
    p = jnp.dot(adj_ref[...], s1_ref[...],
                preferred_element_type=jnp.float32)
    h = jnp.maximum(p + b1_ref[...], 0.0)
    s2_ref[...] = jnp.dot(h, w2_ref[...],
                          preferred_element_type=jnp.float32)


def _pass_c(adj_ref, s2_ref, b2_ref, out_ref):
    z = jnp.dot(adj_ref[...], s2_ref[...],
                preferred_element_type=jnp.float32) + b2_ref[...]
    m = jnp.max(z, axis=1, keepdims=True)
    lse = jnp.log(jnp.sum(jnp.exp(z - m), axis=1, keepdims=True)) + m
    out_ref[...] = z - lse


def kernel(data, adj, W1, b1, W2, b2):
    in_feat = data.shape[1]
    hid = W1.shape[1]
    nout = W2.shape[1]
    b1r = b1.reshape(1, hid)
    b2r = b2.reshape(1, nout)
    grid = (N // TM,)

    s1 = pl.pallas_call(
        _pass_a,
        grid=grid,
        in_specs=[
            pl.BlockSpec((TM, in_feat), lambda i: (i, 0)),
            pl.BlockSpec((in_feat, hid), lambda i: (0, 0)),
        ],
        out_specs=pl.BlockSpec((TM, hid), lambda i: (i, 0)),
        out_shape=jax.ShapeDtypeStruct((N, hid), jnp.float32),
        compiler_params=pltpu.CompilerParams(
            dimension_semantics=("arbitrary",)),
    )(data, W1)

    s2 = pl.pallas_call(
        _pass_b,
        grid=grid,
        in_specs=[
            pl.BlockSpec((TM, N), lambda i: (i, 0)),
            pl.BlockSpec((N, hid), lambda i: (0, 0)),
            pl.BlockSpec((1, hid), lambda i: (0, 0)),
            pl.BlockSpec((hid, nout), lambda i: (0, 0)),
        ],
        out_specs=pl.BlockSpec((TM, nout), lambda i: (i, 0)),
        out_shape=jax.ShapeDtypeStruct((N, nout), jnp.float32),
        compiler_params=pltpu.CompilerParams(
            dimension_semantics=("arbitrary",)),
    )(adj, s1, b1r, W2)

    out = pl.pallas_call(
        _pass_c,
        grid=grid,
        in_specs=[
            pl.BlockSpec((TM, N), lambda i: (i, 0)),
            pl.BlockSpec((N, nout), lambda i: (0, 0)),
            pl.BlockSpec((1, nout), lambda i: (0, 0)),
        ],
        out_specs=pl.BlockSpec((TM, nout), lambda i: (i, 0)),
        out_shape=jax.ShapeDtypeStruct((N, nout), jnp.float32),
        compiler_params=pltpu.CompilerParams(
            dimension_semantics=("arbitrary",)),
    )(adj, s2, b2r)

    return out
